# baseline (speedup 1.0000x reference)
import functools

import jax
import jax.numpy as jnp
from jax import lax
from jax.experimental import pallas as pl
from jax.experimental.pallas import tpu as pltpu
from jax.experimental.pallas import tpu_sc as plsc

N = 10000
E = 320000
D = 128
DOUT = 64
RW = 0.7

NC, NS = 2, 16
NW = NC * NS
NPAD = 10240
NPT = NPAD // NS
NPW = NPAD // NW
ER = 2560
EPW = ER // NW
EPW0 = 120
EPW1 = 40
EN0 = 160
KS = 4
EPAD = ER * 128
F32 = jnp.float32

BLK = 512


def _mesh():
    return plsc.VectorSubcoreMesh(core_axis_name="c", subcore_axis_name="s")


def _wid():
    return lax.axis_index("s") * NC + lax.axis_index("c")


def _sc_deg(colr, colbr, ewr):
    @functools.partial(
        pl.kernel, mesh=_mesh(),
        out_type=(jax.ShapeDtypeStruct((NC, NPAD), F32),
                  jax.ShapeDtypeStruct((NC, NPAD), F32)),
        scratch_types=[
            pltpu.VMEM((EPW0, 128), jnp.int32),
            pltpu.VMEM((EPW0, 128), jnp.int32),
            pltpu.VMEM((EPW0, 128), F32),
            pltpu.VMEM((NPT,), F32),
            pltpu.VMEM_SHARED((NPAD,), F32),
            pltpu.VMEM_SHARED((NPAD,), F32),
            pltpu.SemaphoreType.DMA,
        ],
    )
    def k(col_h, colb_h, ew_h, dg_h, dgb_h, colbuf, colbbuf, ewbuf,
          z_v, acc, accb, sem):
        c = lax.axis_index("c")
        s = lax.axis_index("s")

        def zz(i, _):
            z_v[pl.ds(i * 16, 16)] = jnp.zeros((16,), F32)
            return 0
        lax.fori_loop(0, NPT // 16, zz, 0)
        stripe = pl.ds(s * NPT, NPT)
        pltpu.sync_copy(z_v, acc.at[stripe])
        pltpu.sync_copy(z_v, accb.at[stripe])
        plsc.subcore_barrier()

        def work(base, n):
            bsl = pl.ds(base, n)
            ssl = pl.ds(0, n)
            pltpu.sync_copy(col_h.at[bsl], colbuf.at[ssl])
            pltpu.sync_copy(colb_h.at[bsl], colbbuf.at[ssl])
            pltpu.sync_copy(ew_h.at[bsl], ewbuf.at[ssl])

            def wave(g, _):
                for cb, ac in ((colbuf, acc), (colbbuf, accb)):
                    descs = [
                        pltpu.async_copy(ewbuf.at[g * 8 + p],
                                         ac.at[cb.at[g * 8 + p]],
                                         sem, add=True)
                        for p in range(8)
                    ]
                    for d in descs:
                        d.wait()
                return 0
            lax.fori_loop(0, n // 8, wave, 0)

        @pl.when(c == 0)
        def _():
            work(s * EPW0, EPW0)

        @pl.when(c == 1)
        def _():
            work(NS * EPW0 + s * EPW1, EPW1)
        plsc.subcore_barrier()
        pltpu.sync_copy(acc.at[stripe], dg_h.at[c, stripe])
        pltpu.sync_copy(accb.at[stripe], dgb_h.at[c, stripe])

    return k(colr, colbr, ewr)


def _sc_norm(rowr, colr, rowbr, colbr, ewr, dinv, dinvb):
    @functools.partial(
        pl.kernel, mesh=_mesh(),
        out_type=(jax.ShapeDtypeStruct((ER, 128), F32),
                  jax.ShapeDtypeStruct((ER, 128), F32)),
        scratch_types=[
            pltpu.VMEM((EPW0, 128), jnp.int32),
            pltpu.VMEM((EPW0, 128), jnp.int32),
            pltpu.VMEM((EPW0, 128), F32),
            pltpu.VMEM((EPW0, 128), F32),
            pltpu.VMEM((KS, 128), F32),
            pltpu.VMEM((KS, 128), F32),
            pltpu.SemaphoreType.DMA,
        ],
    )
    def k(row_h, col_h, rowb_h, colb_h, ew_h, dinv_h, dinvb_h,
          norm_h, normb_h, rbuf, cbuf, ebuf, obuf, a_s, b_s, sem):
        c = lax.axis_index("c")
        s = lax.axis_index("s")

        def work(base, n):
            bsl = pl.ds(base, n)
            ssl = pl.ds(0, n)
            pltpu.sync_copy(ew_h.at[bsl], ebuf.at[ssl])
            for (rh, ch, dref, oh) in ((row_h, col_h, dinv_h, norm_h),
                                       (rowb_h, colb_h, dinvb_h, normb_h)):
                pltpu.sync_copy(rh.at[bsl], rbuf.at[ssl])
                pltpu.sync_copy(ch.at[bsl], cbuf.at[ssl])

                def group(g, _):
                    descs = []
                    for p in range(KS):
                        i = g * KS + p
                        descs.append(pltpu.async_copy(dref.at[rbuf.at[i]],
                                                      a_s.at[p], sem))
                        descs.append(pltpu.async_copy(dref.at[cbuf.at[i]],
                                                      b_s.at[p], sem))
                    for d in descs:
                        d.wait()
                    for p in range(KS):
                        i = g * KS + p
                        for j in range(8):
                            sl = pl.ds(j * 16, 16)
                            obuf[i, sl] = a_s[p, sl] * ebuf[i, sl] * b_s[p, sl]
                    return 0
                lax.fori_loop(0, n // KS, group, 0)
                pltpu.sync_copy(obuf.at[ssl], oh.at[bsl])

        @pl.when(c == 0)
        def _():
            work(s * EPW0, EPW0)

        @pl.when(c == 1)
        def _():
            work(NS * EPW0 + s * EPW1, EPW1)

    return k(rowr, colr, rowbr, colbr, ewr, dinv, dinvb)


def _sc_permblend(src, permp, lamv):
    @functools.partial(
        pl.kernel, mesh=_mesh(),
        out_type=(jax.ShapeDtypeStruct((NPAD, D), F32),
                  jax.ShapeDtypeStruct((NPAD, D), F32)),
        scratch_types=[
            pltpu.VMEM((NPW,), jnp.int32),
            pltpu.VMEM((NPW, D), F32),
            pltpu.VMEM((NPW, D), F32),
            pltpu.VMEM((16,), F32),
            pltpu.SemaphoreType.DMA,
        ],
    )
    def k(src_h, perm_h, lam_h, g_out, mix_out, idx_v, g_v, a_v, lam_v, sem):
        w = _wid()
        base = w * NPW
        pltpu.sync_copy(lam_h, lam_v)
        pltpu.sync_copy(perm_h.at[pl.ds(base, NPW)], idx_v)
        pltpu.async_copy(src_h.at[idx_v], g_v, sem).wait()
        pltpu.sync_copy(src_h.at[pl.ds(base, NPW)], a_v)
        lv = lam_v[...]
        om = 1.0 - lv

        def rloop(i, _):
            for j in range(D // 16):
                sl = pl.ds(j * 16, 16)
                a_v[i, sl] = lv * a_v[i, sl] + om * g_v[i, sl]
            return 0
        lax.fori_loop(0, NPW, rloop, 0)
        pltpu.sync_copy(g_v, g_out.at[pl.ds(base, NPW)])
        pltpu.sync_copy(a_v, mix_out.at[pl.ds(base, NPW)])

    return k(src, permp, lamv)


def _sc_perm(src, permp):
    @functools.partial(
        pl.kernel, mesh=_mesh(),
        out_type=jax.ShapeDtypeStruct((NPAD, D), F32),
        scratch_types=[
            pltpu.VMEM((NPW,), jnp.int32),
            pltpu.VMEM((NPW, D), F32),
            pltpu.SemaphoreType.DMA,
        ],
    )
    def k(src_h, perm_h, g_out, idx_v, g_v, sem):
        w = _wid()
        base = w * NPW
        pltpu.sync_copy(perm_h.at[pl.ds(base, NPW)], idx_v)
        pltpu.async_copy(src_h.at[idx_v], g_v, sem).wait()
        pltpu.sync_copy(g_v, g_out.at[pl.ds(base, NPW)])

    return k(src, permp)


def _sc_edge(table, rowr, colr, coefr):
    @functools.partial(
        pl.kernel, mesh=_mesh(),
        out_type=jax.ShapeDtypeStruct((1, NPAD, D), F32),
        scratch_types=[
            pltpu.VMEM((2, 128), jnp.int32),
            pltpu.VMEM((2, 128), jnp.int32),
            pltpu.VMEM((2, 128), F32),
            pltpu.VMEM((2, 128, D), F32),
            pltpu.VMEM_SHARED((NPAD, D), F32),
            pltpu.SemaphoreType.DMA,
            pltpu.SemaphoreType.DMA,
            pltpu.SemaphoreType.DMA,
            pltpu.SemaphoreType.DMA,
        ],
    )
    def k(tab_h, row_h, col_h, cf_h, out_h, ridx, cidx, cfs, rows,
          acc, gsem0, gsem1, ssem0, ssem1):
        c = lax.axis_index("c")
        s = lax.axis_index("s")
        gsem = (gsem0, gsem1)
        ssem = (ssem0, ssem1)

        def zrow(i, _):
            for j in range(D // 16):
                rows[0, i, pl.ds(j * 16, 16)] = jnp.zeros((16,), F32)
            return 0
        lax.fori_loop(0, 128, zrow, 0)

        @pl.when(c == 0)
        def _():
            def zcp(kk, _):
                pltpu.sync_copy(rows.at[0],
                                acc.at[pl.ds(s * NPT + kk * 128, 128)])
                return 0
            lax.fori_loop(0, NPT // 128, zcp, 0)
        plsc.subcore_barrier()

        def work(base, n):
            def fire_gather(i, p):
                pltpu.sync_copy(row_h.at[base + i], ridx.at[p])
                pltpu.sync_copy(col_h.at[base + i], cidx.at[p])
                pltpu.sync_copy(cf_h.at[base + i], cfs.at[p])
                return pltpu.async_copy(tab_h.at[ridx.at[p]], rows.at[p],
                                        gsem[p])

            def wait_gather(p):
                pltpu.make_async_copy(tab_h.at[ridx.at[p]], rows.at[p],
                                      gsem[p]).wait()

            def fire_scatter(i, p):
                return pltpu.async_copy(rows.at[p], acc.at[cidx.at[p]],
                                        ssem[p], add=True)

            def wait_scatter(i, p):
                pltpu.make_async_copy(rows.at[p], acc.at[cidx.at[p]],
                                      ssem[p]).wait()

            def scale(p, _):
                def sc(gg, _):
                    cvec = cfs[p, pl.ds(gg * 16, 16)]
                    for b16 in range(16):
                        cf = cvec[b16]
                        b = gg * 16 + b16
                        for j in range(D // 16):
                            sl = pl.ds(j * 16, 16)
                            rows[p, b, sl] = rows[p, b, sl] * cf
                    return 0
                lax.fori_loop(0, 8, sc, 0)

            fire_gather(0, 0)
            fire_gather(1, 1)
            wait_gather(0)
            scale(0, None)
            fire_scatter(0, 0)

            def step(i, p):
                q = 1 - p
                wait_scatter(i - 1, q)
                fire_gather(i + 1, q)
                wait_gather(p)
                scale(p, None)
                fire_scatter(i, p)

            def pair(g, _):
                step(2 * g + 1, 1)
                step(2 * g + 2, 0)
                return 0
            lax.fori_loop(0, (n - 2) // 2, pair, 0)
            wait_scatter(n - 2, 0)
            wait_gather(1)
            scale(1, None)
            fire_scatter(n - 1, 1)
            wait_scatter(n - 1, 1)

        @pl.when(c == 0)
        def _():
            work(s * EN0, EN0)
        plsc.subcore_barrier()

        @pl.when(c == 0)
        def _():
            stripe = pl.ds(s * NPT, NPT)
            pltpu.sync_copy(acc.at[stripe], out_h.at[0, stripe])

    return k(table, rowr, colr, coefr)


def _tc_dinv(dg, dgb):
    def body(dg_ref, dgb_ref, o1_ref, o2_ref):
        for dref, oref in ((dg_ref, o1_ref), (dgb_ref, o2_ref)):
            d = dref[0] + dref[1]
            safe = jnp.where(d > 0, d, 1.0)
            oref[...] = jnp.where(d > 0, 1.0 / jnp.sqrt(safe), 0.0)

    nr = NPAD // 128
    return pl.pallas_call(
        body,
        out_shape=(jax.ShapeDtypeStruct((nr, 128), F32),
                   jax.ShapeDtypeStruct((nr, 128), F32)),
    )(dg.reshape(NC, nr, 128), dgb.reshape(NC, nr, 128))


def _tc_lin(x, W, b):
    K = W.shape[1]

    def body(x_ref, w_ref, b_ref, o_ref):
        o_ref[...] = jnp.dot(x_ref[...], w_ref[...],
                             preferred_element_type=F32) + b_ref[...]

    return pl.pallas_call(
        body,
        grid=(NPAD // BLK,),
        in_specs=[pl.BlockSpec((BLK, D), lambda i: (i, 0)),
                  pl.BlockSpec((D, K), lambda i: (0, 0)),
                  pl.BlockSpec((1, K), lambda i: (0, 0))],
        out_specs=pl.BlockSpec((BLK, K), lambda i: (i, 0)),
        out_shape=jax.ShapeDtypeStruct((NPAD, K), F32),
    )(x, W, b)


def _tc_mid(agg1, agg3, agg4, H0, H0p, W2, b2, lam2):
    def body(a1, a3, a4, h0, h0p, w2, b2r, lamr, h1_o, hm1_o):
        lam = lamr[0, 0]
        h0v = h0[...]
        x1 = jnp.maximum(RW * a1[0] + (1.0 - RW) * h0v, 0.0)
        n1 = jnp.maximum(RW * a3[0] + (1.0 - RW) * h0v, 0.0)
        n1b = jnp.maximum(RW * a4[0] + (1.0 - RW) * h0p[...], 0.0)
        xm = lam * n1 + (1.0 - lam) * n1b
        h1_o[...] = jnp.dot(x1, w2[...], preferred_element_type=F32) + b2r[...]
        hm1_o[...] = jnp.dot(xm, w2[...], preferred_element_type=F32) + b2r[...]

    aspec = pl.BlockSpec((1, BLK, D), lambda i: (0, i, 0))
    hspec = pl.BlockSpec((BLK, D), lambda i: (i, 0))
    return pl.pallas_call(
        body,
        grid=(NPAD // BLK,),
        in_specs=[aspec, aspec, aspec, hspec, hspec,
                  pl.BlockSpec((D, D), lambda i: (0, 0)),
                  pl.BlockSpec((1, D), lambda i: (0, 0)),
                  pl.BlockSpec((1, 1), lambda i: (0, 0))],
        out_specs=(hspec, hspec),
        out_shape=(jax.ShapeDtypeStruct((NPAD, D), F32),
                   jax.ShapeDtypeStruct((NPAD, D), F32)),
    )(agg1, agg3, agg4, H0, H0p, W2, b2, lam2)


def _tc_fin(agg5, agg6, H1, H1p, Wl, bl, lam2):
    def body(a5, a6, h1, h1p, wl, blr, lamr, o_ref):
        lam = lamr[0, 0]
        n2 = jnp.maximum(RW * a5[0] + (1.0 - RW) * h1[...], 0.0)
        n2b = jnp.maximum(RW * a6[0] + (1.0 - RW) * h1p[...], 0.0)
        xm = lam * n2 + (1.0 - lam) * n2b
        o = jnp.dot(xm, wl[...], preferred_element_type=F32) + blr[...]
        m = jnp.max(o, axis=-1, keepdims=True)
        lse = jnp.log(jnp.sum(jnp.exp(o - m), axis=-1, keepdims=True))
        o_ref[...] = o - m - lse

    aspec = pl.BlockSpec((1, BLK, D), lambda i: (0, i, 0))
    hspec = pl.BlockSpec((BLK, D), lambda i: (i, 0))
    return pl.pallas_call(
        body,
        grid=(NPAD // BLK,),
        in_specs=[aspec, aspec, hspec, hspec,
                  pl.BlockSpec((D, DOUT), lambda i: (0, 0)),
                  pl.BlockSpec((1, DOUT), lambda i: (0, 0)),
                  pl.BlockSpec((1, 1), lambda i: (0, 0))],
        out_specs=pl.BlockSpec((BLK, DOUT), lambda i: (i, 0)),
        out_shape=jax.ShapeDtypeStruct((NPAD, DOUT), F32),
    )(agg5, agg6, H1, H1p, Wl, bl, lam2)


def kernel(x0, edge_index, edge_index_b, lam, id_new_value_old, edge_weight,
           W1, b1, W2, b2, Wl, bl):
    x0p = jnp.zeros((NPAD, D), F32).at[:N].set(x0)

    def pad_i(a):
        return jnp.concatenate(
            [a.astype(jnp.int32), jnp.zeros((EPAD - E,), jnp.int32)]
        ).reshape(ER, 128)

    rowr, colr = pad_i(edge_index[0]), pad_i(edge_index[1])
    rowbr, colbr = pad_i(edge_index_b[0]), pad_i(edge_index_b[1])
    ewr = jnp.concatenate(
        [edge_weight.astype(F32), jnp.zeros((EPAD - E,), F32)]).reshape(ER, 128)
    permp = jnp.concatenate(
        [id_new_value_old.astype(jnp.int32), jnp.zeros((NPAD - N,), jnp.int32)])
    lamv = jnp.full((16,), lam, F32)
    lam2 = jnp.reshape(lam, (1, 1)).astype(F32)

    dg, dgb = _sc_deg(colr, colbr, ewr)
    dinv2, dinvb2 = _tc_dinv(dg, dgb)
    dinv, dinvb = dinv2.reshape(NPAD), dinvb2.reshape(NPAD)
    normr, normbr = _sc_norm(rowr, colr, rowbr, colbr, ewr, dinv, dinvb)

    H0 = _tc_lin(x0p, W1, b1.reshape(1, D))
    H0p, Hmix0 = _sc_permblend(H0, permp, lamv)

    agg1 = _sc_edge(H0, rowr, colr, normr)
    agg3 = _sc_edge(Hmix0, rowr, colr, normr)
    agg4 = _sc_edge(Hmix0, rowbr, colbr, normbr)

    H1, Hmix1 = _tc_mid(agg1, agg3, agg4, H0, H0p, W2, b2.reshape(1, D), lam2)
    H1p = _sc_perm(H1, permp)

    agg5 = _sc_edge(Hmix1, rowr, colr, normr)
    agg6 = _sc_edge(Hmix1, rowbr, colbr, normbr)

    out = _tc_fin(agg5, agg6, H1, H1p, Wl, bl.reshape(1, DOUT), lam2)
    return out[:N]


# speedup vs baseline: 1.3760x; 1.3760x over previous
import functools

import jax
import jax.numpy as jnp
from jax import lax
from jax.experimental import pallas as pl
from jax.experimental.pallas import tpu as pltpu
from jax.experimental.pallas import tpu_sc as plsc

N = 10000
E = 320000
D = 128
DOUT = 64
RW = 0.7

NC, NS = 2, 16
NW = NC * NS
NPAD = 10240
NPT = NPAD // NS
NPW = NPAD // NW
ER = 2560
EPW = ER // NW
EPW0 = 120
EPW1 = 40
EN0 = 128
EN1 = 32
KS = 4
EPAD = ER * 128
F32 = jnp.float32

BLK = 512


def _mesh():
    return plsc.VectorSubcoreMesh(core_axis_name="c", subcore_axis_name="s")


def _wid():
    return lax.axis_index("s") * NC + lax.axis_index("c")


def _sc_deg(colr, colbr, ewr):
    @functools.partial(
        pl.kernel, mesh=_mesh(),
        out_type=(jax.ShapeDtypeStruct((NC, NPAD), F32),
                  jax.ShapeDtypeStruct((NC, NPAD), F32)),
        scratch_types=[
            pltpu.VMEM((EPW0, 128), jnp.int32),
            pltpu.VMEM((EPW0, 128), jnp.int32),
            pltpu.VMEM((EPW0, 128), F32),
            pltpu.VMEM((NPT,), F32),
            pltpu.VMEM_SHARED((NPAD,), F32),
            pltpu.VMEM_SHARED((NPAD,), F32),
            pltpu.SemaphoreType.DMA,
        ],
    )
    def k(col_h, colb_h, ew_h, dg_h, dgb_h, colbuf, colbbuf, ewbuf,
          z_v, acc, accb, sem):
        c = lax.axis_index("c")
        s = lax.axis_index("s")

        def zz(i, _):
            z_v[pl.ds(i * 16, 16)] = jnp.zeros((16,), F32)
            return 0
        lax.fori_loop(0, NPT // 16, zz, 0)
        stripe = pl.ds(s * NPT, NPT)
        pltpu.sync_copy(z_v, acc.at[stripe])
        pltpu.sync_copy(z_v, accb.at[stripe])
        plsc.subcore_barrier()

        def work(base, n):
            bsl = pl.ds(base, n)
            ssl = pl.ds(0, n)
            pltpu.sync_copy(col_h.at[bsl], colbuf.at[ssl])
            pltpu.sync_copy(colb_h.at[bsl], colbbuf.at[ssl])
            pltpu.sync_copy(ew_h.at[bsl], ewbuf.at[ssl])

            def wave(g, _):
                for cb, ac in ((colbuf, acc), (colbbuf, accb)):
                    descs = [
                        pltpu.async_copy(ewbuf.at[g * 8 + p],
                                         ac.at[cb.at[g * 8 + p]],
                                         sem, add=True)
                        for p in range(8)
                    ]
                    for d in descs:
                        d.wait()
                return 0
            lax.fori_loop(0, n // 8, wave, 0)

        @pl.when(c == 0)
        def _():
            work(s * EPW0, EPW0)

        @pl.when(c == 1)
        def _():
            work(NS * EPW0 + s * EPW1, EPW1)
        plsc.subcore_barrier()
        pltpu.sync_copy(acc.at[stripe], dg_h.at[c, stripe])
        pltpu.sync_copy(accb.at[stripe], dgb_h.at[c, stripe])

    return k(colr, colbr, ewr)


def _sc_norm(rowr, colr, rowbr, colbr, ewr, dinv, dinvb):
    @functools.partial(
        pl.kernel, mesh=_mesh(),
        out_type=(jax.ShapeDtypeStruct((ER, 128), F32),
                  jax.ShapeDtypeStruct((ER, 128), F32)),
        scratch_types=[
            pltpu.VMEM((EPW0, 128), jnp.int32),
            pltpu.VMEM((EPW0, 128), jnp.int32),
            pltpu.VMEM((EPW0, 128), F32),
            pltpu.VMEM((EPW0, 128), F32),
            pltpu.VMEM((KS, 128), F32),
            pltpu.VMEM((KS, 128), F32),
            pltpu.SemaphoreType.DMA,
        ],
    )
    def k(row_h, col_h, rowb_h, colb_h, ew_h, dinv_h, dinvb_h,
          norm_h, normb_h, rbuf, cbuf, ebuf, obuf, a_s, b_s, sem):
        c = lax.axis_index("c")
        s = lax.axis_index("s")

        def work(base, n):
            bsl = pl.ds(base, n)
            ssl = pl.ds(0, n)
            pltpu.sync_copy(ew_h.at[bsl], ebuf.at[ssl])
            for (rh, ch, dref, oh) in ((row_h, col_h, dinv_h, norm_h),
                                       (rowb_h, colb_h, dinvb_h, normb_h)):
                pltpu.sync_copy(rh.at[bsl], rbuf.at[ssl])
                pltpu.sync_copy(ch.at[bsl], cbuf.at[ssl])

                def group(g, _):
                    descs = []
                    for p in range(KS):
                        i = g * KS + p
                        descs.append(pltpu.async_copy(dref.at[rbuf.at[i]],
                                                      a_s.at[p], sem))
                        descs.append(pltpu.async_copy(dref.at[cbuf.at[i]],
                                                      b_s.at[p], sem))
                    for d in descs:
                        d.wait()
                    for p in range(KS):
                        i = g * KS + p
                        for j in range(8):
                            sl = pl.ds(j * 16, 16)
                            obuf[i, sl] = a_s[p, sl] * ebuf[i, sl] * b_s[p, sl]
                    return 0
                lax.fori_loop(0, n // KS, group, 0)
                pltpu.sync_copy(obuf.at[ssl], oh.at[bsl])

        @pl.when(c == 0)
        def _():
            work(s * EPW0, EPW0)

        @pl.when(c == 1)
        def _():
            work(NS * EPW0 + s * EPW1, EPW1)

    return k(rowr, colr, rowbr, colbr, ewr, dinv, dinvb)


def _sc_permblend(src, permp, lamv):
    @functools.partial(
        pl.kernel, mesh=_mesh(),
        out_type=(jax.ShapeDtypeStruct((NPAD, D), F32),
                  jax.ShapeDtypeStruct((NPAD, D), F32)),
        scratch_types=[
            pltpu.VMEM((NPW,), jnp.int32),
            pltpu.VMEM((NPW, D), F32),
            pltpu.VMEM((NPW, D), F32),
            pltpu.VMEM((16,), F32),
            pltpu.SemaphoreType.DMA,
        ],
    )
    def k(src_h, perm_h, lam_h, g_out, mix_out, idx_v, g_v, a_v, lam_v, sem):
        w = _wid()
        base = w * NPW
        pltpu.sync_copy(lam_h, lam_v)
        pltpu.sync_copy(perm_h.at[pl.ds(base, NPW)], idx_v)
        pltpu.async_copy(src_h.at[idx_v], g_v, sem).wait()
        pltpu.sync_copy(src_h.at[pl.ds(base, NPW)], a_v)
        lv = lam_v[...]
        om = 1.0 - lv

        def rloop(i, _):
            for j in range(D // 16):
                sl = pl.ds(j * 16, 16)
                a_v[i, sl] = lv * a_v[i, sl] + om * g_v[i, sl]
            return 0
        lax.fori_loop(0, NPW, rloop, 0)
        pltpu.sync_copy(g_v, g_out.at[pl.ds(base, NPW)])
        pltpu.sync_copy(a_v, mix_out.at[pl.ds(base, NPW)])

    return k(src, permp, lamv)


def _sc_perm(src, permp):
    @functools.partial(
        pl.kernel, mesh=_mesh(),
        out_type=jax.ShapeDtypeStruct((NPAD, D), F32),
        scratch_types=[
            pltpu.VMEM((NPW,), jnp.int32),
            pltpu.VMEM((NPW, D), F32),
            pltpu.SemaphoreType.DMA,
        ],
    )
    def k(src_h, perm_h, g_out, idx_v, g_v, sem):
        w = _wid()
        base = w * NPW
        pltpu.sync_copy(perm_h.at[pl.ds(base, NPW)], idx_v)
        pltpu.async_copy(src_h.at[idx_v], g_v, sem).wait()
        pltpu.sync_copy(g_v, g_out.at[pl.ds(base, NPW)])

    return k(src, permp)


def _sc_edge(table, rowr, colr, coefr):
    @functools.partial(
        pl.kernel, mesh=_mesh(),
        out_type=jax.ShapeDtypeStruct((NC, NPAD, D), F32),
        scratch_types=[
            pltpu.VMEM((2, 128), jnp.int32),
            pltpu.VMEM((2, 128), jnp.int32),
            pltpu.VMEM((2, 128), F32),
            pltpu.VMEM((2, 128, D), F32),
            pltpu.VMEM_SHARED((NPAD, D), F32),
            pltpu.SemaphoreType.DMA,
            pltpu.SemaphoreType.DMA,
            pltpu.SemaphoreType.DMA,
            pltpu.SemaphoreType.DMA,
        ],
    )
    def k(tab_h, row_h, col_h, cf_h, out_h, ridx, cidx, cfs, rows,
          acc, gsem0, gsem1, ssem0, ssem1):
        c = lax.axis_index("c")
        s = lax.axis_index("s")
        gsem = (gsem0, gsem1)
        ssem = (ssem0, ssem1)

        def zrow(i, _):
            for j in range(D // 16):
                rows[0, i, pl.ds(j * 16, 16)] = jnp.zeros((16,), F32)
            return 0
        lax.fori_loop(0, 128, zrow, 0)

        def zcp(kk, _):
            pltpu.sync_copy(rows.at[0],
                            acc.at[pl.ds(s * NPT + kk * 128, 128)])
            return 0
        lax.fori_loop(0, NPT // 128, zcp, 0)
        plsc.subcore_barrier()

        def work(base, n):
            def fire_gather(i, p):
                pltpu.sync_copy(row_h.at[base + i], ridx.at[p])
                pltpu.sync_copy(col_h.at[base + i], cidx.at[p])
                pltpu.sync_copy(cf_h.at[base + i], cfs.at[p])
                return pltpu.async_copy(tab_h.at[ridx.at[p]], rows.at[p],
                                        gsem[p])

            def wait_gather(p):
                pltpu.make_async_copy(tab_h.at[ridx.at[p]], rows.at[p],
                                      gsem[p]).wait()

            def fire_scatter(i, p):
                return pltpu.async_copy(rows.at[p], acc.at[cidx.at[p]],
                                        ssem[p], add=True)

            def wait_scatter(i, p):
                pltpu.make_async_copy(rows.at[p], acc.at[cidx.at[p]],
                                      ssem[p]).wait()

            def scale(p, _):
                def sc(gg, _):
                    cvec = cfs[p, pl.ds(gg * 16, 16)]
                    for b16 in range(16):
                        cf = cvec[b16]
                        b = gg * 16 + b16
                        for j in range(D // 16):
                            sl = pl.ds(j * 16, 16)
                            rows[p, b, sl] = rows[p, b, sl] * cf
                    return 0
                lax.fori_loop(0, 8, sc, 0)

            fire_gather(0, 0)
            fire_gather(1, 1)
            wait_gather(0)
            scale(0, None)
            fire_scatter(0, 0)

            def step(i, p):
                q = 1 - p
                wait_scatter(i - 1, q)
                fire_gather(i + 1, q)
                wait_gather(p)
                scale(p, None)
                fire_scatter(i, p)

            def pair(g, _):
                step(2 * g + 1, 1)
                step(2 * g + 2, 0)
                return 0
            lax.fori_loop(0, (n - 2) // 2, pair, 0)
            wait_scatter(n - 2, 0)
            wait_gather(1)
            scale(1, None)
            fire_scatter(n - 1, 1)
            wait_scatter(n - 1, 1)

        @pl.when(c == 0)
        def _():
            work(s * EN0, EN0)

        @pl.when(c == 1)
        def _():
            work(NS * EN0 + s * EN1, EN1)
        plsc.subcore_barrier()
        stripe = pl.ds(s * NPT, NPT)
        pltpu.sync_copy(acc.at[stripe], out_h.at[c, stripe])

    return k(table, rowr, colr, coefr)


def _tc_dinv(dg, dgb):
    def body(dg_ref, dgb_ref, o1_ref, o2_ref):
        for dref, oref in ((dg_ref, o1_ref), (dgb_ref, o2_ref)):
            d = dref[0] + dref[1]
            safe = jnp.where(d > 0, d, 1.0)
            oref[...] = jnp.where(d > 0, 1.0 / jnp.sqrt(safe), 0.0)

    nr = NPAD // 128
    return pl.pallas_call(
        body,
        out_shape=(jax.ShapeDtypeStruct((nr, 128), F32),
                   jax.ShapeDtypeStruct((nr, 128), F32)),
    )(dg.reshape(NC, nr, 128), dgb.reshape(NC, nr, 128))


def _tc_lin(x, W, b):
    K = W.shape[1]

    def body(x_ref, w_ref, b_ref, o_ref):
        o_ref[...] = jnp.dot(x_ref[...], w_ref[...],
                             preferred_element_type=F32) + b_ref[...]

    return pl.pallas_call(
        body,
        grid=(NPAD // BLK,),
        in_specs=[pl.BlockSpec((BLK, D), lambda i: (i, 0)),
                  pl.BlockSpec((D, K), lambda i: (0, 0)),
                  pl.BlockSpec((1, K), lambda i: (0, 0))],
        out_specs=pl.BlockSpec((BLK, K), lambda i: (i, 0)),
        out_shape=jax.ShapeDtypeStruct((NPAD, K), F32),
    )(x, W, b)


def _tc_mid(agg1, agg3, agg4, H0, H0p, W2, b2, lam2):
    def body(a1, a3, a4, h0, h0p, w2, b2r, lamr, h1_o, hm1_o):
        lam = lamr[0, 0]
        h0v = h0[...]
        x1 = jnp.maximum(RW * (a1[0] + a1[1]) + (1.0 - RW) * h0v, 0.0)
        n1 = jnp.maximum(RW * (a3[0] + a3[1]) + (1.0 - RW) * h0v, 0.0)
        n1b = jnp.maximum(RW * (a4[0] + a4[1]) + (1.0 - RW) * h0p[...], 0.0)
        xm = lam * n1 + (1.0 - lam) * n1b
        h1_o[...] = jnp.dot(x1, w2[...], preferred_element_type=F32) + b2r[...]
        hm1_o[...] = jnp.dot(xm, w2[...], preferred_element_type=F32) + b2r[...]

    aspec = pl.BlockSpec((1, BLK, D), lambda i: (0, i, 0))
    hspec = pl.BlockSpec((BLK, D), lambda i: (i, 0))
    return pl.pallas_call(
        body,
        grid=(NPAD // BLK,),
        in_specs=[aspec, aspec, aspec, hspec, hspec,
                  pl.BlockSpec((D, D), lambda i: (0, 0)),
                  pl.BlockSpec((1, D), lambda i: (0, 0)),
                  pl.BlockSpec((1, 1), lambda i: (0, 0))],
        out_specs=(hspec, hspec),
        out_shape=(jax.ShapeDtypeStruct((NPAD, D), F32),
                   jax.ShapeDtypeStruct((NPAD, D), F32)),
    )(agg1, agg3, agg4, H0, H0p, W2, b2, lam2)


def _tc_fin(agg5, agg6, H1, H1p, Wl, bl, lam2):
    def body(a5, a6, h1, h1p, wl, blr, lamr, o_ref):
        lam = lamr[0, 0]
        n2 = jnp.maximum(RW * (a5[0] + a5[1]) + (1.0 - RW) * h1[...], 0.0)
        n2b = jnp.maximum(RW * (a6[0] + a6[1]) + (1.0 - RW) * h1p[...], 0.0)
        xm = lam * n2 + (1.0 - lam) * n2b
        o = jnp.dot(xm, wl[...], preferred_element_type=F32) + blr[...]
        m = jnp.max(o, axis=-1, keepdims=True)
        lse = jnp.log(jnp.sum(jnp.exp(o - m), axis=-1, keepdims=True))
        o_ref[...] = o - m - lse

    aspec = pl.BlockSpec((1, BLK, D), lambda i: (0, i, 0))
    hspec = pl.BlockSpec((BLK, D), lambda i: (i, 0))
    return pl.pallas_call(
        body,
        grid=(NPAD // BLK,),
        in_specs=[aspec, aspec, hspec, hspec,
                  pl.BlockSpec((D, DOUT), lambda i: (0, 0)),
                  pl.BlockSpec((1, DOUT), lambda i: (0, 0)),
                  pl.BlockSpec((1, 1), lambda i: (0, 0))],
        out_specs=pl.BlockSpec((BLK, DOUT), lambda i: (i, 0)),
        out_shape=jax.ShapeDtypeStruct((NPAD, DOUT), F32),
    )(agg5, agg6, H1, H1p, Wl, bl, lam2)


def kernel(x0, edge_index, edge_index_b, lam, id_new_value_old, edge_weight,
           W1, b1, W2, b2, Wl, bl):
    x0p = jnp.zeros((NPAD, D), F32).at[:N].set(x0)

    def pad_i(a):
        return jnp.concatenate(
            [a.astype(jnp.int32), jnp.zeros((EPAD - E,), jnp.int32)]
        ).reshape(ER, 128)

    rowr, colr = pad_i(edge_index[0]), pad_i(edge_index[1])
    rowbr, colbr = pad_i(edge_index_b[0]), pad_i(edge_index_b[1])
    ewr = jnp.concatenate(
        [edge_weight.astype(F32), jnp.zeros((EPAD - E,), F32)]).reshape(ER, 128)
    permp = jnp.concatenate(
        [id_new_value_old.astype(jnp.int32), jnp.zeros((NPAD - N,), jnp.int32)])
    lamv = jnp.full((16,), lam, F32)
    lam2 = jnp.reshape(lam, (1, 1)).astype(F32)

    dg, dgb = _sc_deg(colr, colbr, ewr)
    dinv2, dinvb2 = _tc_dinv(dg, dgb)
    dinv, dinvb = dinv2.reshape(NPAD), dinvb2.reshape(NPAD)
    normr, normbr = _sc_norm(rowr, colr, rowbr, colbr, ewr, dinv, dinvb)

    H0 = _tc_lin(x0p, W1, b1.reshape(1, D))
    H0p, Hmix0 = _sc_permblend(H0, permp, lamv)

    agg1 = _sc_edge(H0, rowr, colr, normr)
    agg3 = _sc_edge(Hmix0, rowr, colr, normr)
    agg4 = _sc_edge(Hmix0, rowbr, colbr, normbr)

    H1, Hmix1 = _tc_mid(agg1, agg3, agg4, H0, H0p, W2, b2.reshape(1, D), lam2)
    H1p = _sc_perm(H1, permp)

    agg5 = _sc_edge(Hmix1, rowr, colr, normr)
    agg6 = _sc_edge(Hmix1, rowbr, colbr, normbr)

    out = _tc_fin(agg5, agg6, H1, H1p, Wl, bl.reshape(1, DOUT), lam2)
    return out[:N]
